# baseline (speedup 1.0000x reference)
import functools
import math

import jax
import jax.numpy as jnp
from jax import lax
from jax.experimental import pallas as pl
from jax.experimental.pallas import tpu as pltpu

_ROW_TILE = 256


def _atten_pool_kernel(x_full_ref, x_tile_ref, segc_ref, segr_ref,
                       wq_ref, bq_ref, wk_ref, bk_ref, wv_ref, bv_ref,
                       ws_ref, bs_ref,
                       out_ref, k_ref, v_ref, *, num_segments, scale):
    i = pl.program_id(0)

    @pl.when(i == 0)
    def _init():
        x_full = x_full_ref[:]
        k = jnp.dot(x_full, wk_ref[:],
                    preferred_element_type=jnp.float32) + bk_ref[:]
        v = jnp.dot(x_full, wv_ref[:],
                    preferred_element_type=jnp.float32) + bv_ref[:]
        k_ref[:] = k.astype(jnp.bfloat16)
        v_ref[:, :v.shape[1]] = v.astype(jnp.bfloat16)
        v_ref[:, v.shape[1]:] = jnp.ones_like(v, jnp.bfloat16)
        out_ref[:] = jnp.full_like(out_ref, -jnp.inf)

    x_t = x_tile_ref[:]
    q = (jnp.dot(x_t, wq_ref[:],
                 preferred_element_type=jnp.float32) + bq_ref[:]) * scale

    s = lax.dot_general(q.astype(jnp.bfloat16), k_ref[:],
                        (((1,), (1,)), ((), ())),
                        preferred_element_type=jnp.float32)
    seg_c = segc_ref[0]
    seg_r = segr_ref[:]
    mask = seg_c == seg_r
    s = jnp.where(mask, s, -jnp.inf)
    m = jnp.max(s, axis=1, keepdims=True)
    p = jnp.exp(s - m)

    c = x_t.shape[1]
    ov = jnp.dot(p.astype(jnp.bfloat16), v_ref[:],
                 preferred_element_type=jnp.float32)
    denom = ov[:, c:c + 1]
    o = ov[:, :c] * (1.0 / denom)
    o = o + jnp.dot(x_t, ws_ref[:],
                    preferred_element_type=jnp.float32) + bs_ref[:]

    first = jnp.min(seg_c)
    last = jnp.max(seg_c)
    for b in range(num_segments):
        @pl.when((b >= first) & (b <= last))
        def _pool():
            mb = seg_c == b
            pb = jnp.max(jnp.where(mb, o, -jnp.inf), axis=0,
                         keepdims=True)
            out_ref[b:b + 1, :] = jnp.maximum(out_ref[b:b + 1, :], pb)


def kernel(x, subgbatch, Wq, bq, Wk, bk, Wv, bv, Wskip, bskip):
    n, d = x.shape
    c = Wq.shape[1]
    num_segments = 16
    t = _ROW_TILE
    num_tiles = n // t
    seg = subgbatch.astype(jnp.int32)
    segc = seg.reshape(num_tiles, t, 1)
    segr = seg.reshape(1, n)

    fn = pl.pallas_call(
        functools.partial(_atten_pool_kernel, num_segments=num_segments,
                          scale=1.0 / math.sqrt(c)),
        grid=(num_tiles,),
        in_specs=[
            pl.BlockSpec((n, d), lambda i: (0, 0)),
            pl.BlockSpec((t, d), lambda i: (i, 0)),
            pl.BlockSpec((1, t, 1), lambda i: (i, 0, 0)),
            pl.BlockSpec((1, n), lambda i: (0, 0)),
            pl.BlockSpec((d, c), lambda i: (0, 0)),
            pl.BlockSpec((1, c), lambda i: (0, 0)),
            pl.BlockSpec((d, c), lambda i: (0, 0)),
            pl.BlockSpec((1, c), lambda i: (0, 0)),
            pl.BlockSpec((d, c), lambda i: (0, 0)),
            pl.BlockSpec((1, c), lambda i: (0, 0)),
            pl.BlockSpec((d, c), lambda i: (0, 0)),
            pl.BlockSpec((1, c), lambda i: (0, 0)),
        ],
        out_specs=pl.BlockSpec((num_segments, c), lambda i: (0, 0)),
        scratch_shapes=[
            pltpu.VMEM((n, c), jnp.bfloat16),
            pltpu.VMEM((n, 2 * c), jnp.bfloat16),
        ],
        out_shape=jax.ShapeDtypeStruct((num_segments, c), jnp.float32),
    )
    return fn(x, x, segc, segr,
              Wq, bq.reshape(1, c), Wk, bk.reshape(1, c),
              Wv, bv.reshape(1, c), Wskip, bskip.reshape(1, c))


# speedup vs baseline: 1.1176x; 1.1176x over previous
import functools
import math

import jax
import jax.numpy as jnp
from jax import lax
from jax.experimental import pallas as pl
from jax.experimental.pallas import tpu as pltpu

_ROW_TILE = 256


def _atten_pool_kernel(x_full_ref, x_tile_ref, segc_ref, segr_ref,
                       wq_ref, bq_ref, wk_ref, bk_ref, wv_ref, bv_ref,
                       ws_ref, bs_ref,
                       out_ref, k_ref, v_ref, *, num_segments, scale):
    i = pl.program_id(0)

    @pl.when(i == 0)
    def _init():
        x_full = x_full_ref[:]
        k = jnp.dot(x_full, wk_ref[:],
                    preferred_element_type=jnp.float32) + bk_ref[:]
        v = jnp.dot(x_full, wv_ref[:],
                    preferred_element_type=jnp.float32) + bv_ref[:]
        k_ref[:] = k.astype(jnp.bfloat16)
        v_ref[:] = v.astype(jnp.bfloat16)
        out_ref[:] = jnp.full_like(out_ref, -jnp.inf)

    x_t = x_tile_ref[:]
    q = (jnp.dot(x_t, wq_ref[:],
                 preferred_element_type=jnp.float32) + bq_ref[:]) * scale

    s = lax.dot_general(q.astype(jnp.bfloat16), k_ref[:],
                        (((1,), (1,)), ((), ())),
                        preferred_element_type=jnp.float32)
    seg_c = segc_ref[0]
    seg_r = segr_ref[:]
    mask = seg_c == seg_r
    s = jnp.where(mask, s, -jnp.inf)
    m = jnp.max(s, axis=1, keepdims=True)
    p = jnp.exp(s - m)
    denom = jnp.sum(p, axis=1, keepdims=True)

    o = jnp.dot(p.astype(jnp.bfloat16), v_ref[:],
                preferred_element_type=jnp.float32) * (1.0 / denom)
    o = o + jnp.dot(x_t, ws_ref[:],
                    preferred_element_type=jnp.float32) + bs_ref[:]

    first = jnp.min(seg_c)
    last = jnp.max(seg_c)
    for b in range(num_segments):
        @pl.when((b >= first) & (b <= last))
        def _pool():
            mb = seg_c == b
            pb = jnp.max(jnp.where(mb, o, -jnp.inf), axis=0,
                         keepdims=True)
            out_ref[b:b + 1, :] = jnp.maximum(out_ref[b:b + 1, :], pb)


def kernel(x, subgbatch, Wq, bq, Wk, bk, Wv, bv, Wskip, bskip):
    n, d = x.shape
    c = Wq.shape[1]
    num_segments = 16
    t = _ROW_TILE
    num_tiles = n // t
    seg = subgbatch.astype(jnp.int32)
    segc = seg.reshape(num_tiles, t, 1)
    segr = seg.reshape(1, n)

    fn = pl.pallas_call(
        functools.partial(_atten_pool_kernel, num_segments=num_segments,
                          scale=1.0 / math.sqrt(c)),
        grid=(num_tiles,),
        in_specs=[
            pl.BlockSpec((n, d), lambda i: (0, 0)),
            pl.BlockSpec((t, d), lambda i: (i, 0)),
            pl.BlockSpec((1, t, 1), lambda i: (i, 0, 0)),
            pl.BlockSpec((1, n), lambda i: (0, 0)),
            pl.BlockSpec((d, c), lambda i: (0, 0)),
            pl.BlockSpec((1, c), lambda i: (0, 0)),
            pl.BlockSpec((d, c), lambda i: (0, 0)),
            pl.BlockSpec((1, c), lambda i: (0, 0)),
            pl.BlockSpec((d, c), lambda i: (0, 0)),
            pl.BlockSpec((1, c), lambda i: (0, 0)),
            pl.BlockSpec((d, c), lambda i: (0, 0)),
            pl.BlockSpec((1, c), lambda i: (0, 0)),
        ],
        out_specs=pl.BlockSpec((num_segments, c), lambda i: (0, 0)),
        scratch_shapes=[
            pltpu.VMEM((n, c), jnp.bfloat16),
            pltpu.VMEM((n, c), jnp.bfloat16),
        ],
        out_shape=jax.ShapeDtypeStruct((num_segments, c), jnp.float32),
    )
    return fn(x, x, segc, segr,
              Wq, bq.reshape(1, c), Wk, bk.reshape(1, c),
              Wv, bv.reshape(1, c), Wskip, bskip.reshape(1, c))


# speedup vs baseline: 1.1218x; 1.0037x over previous
import functools
import math

import jax
import jax.numpy as jnp
from jax import lax
from jax.experimental import pallas as pl
from jax.experimental.pallas import tpu as pltpu

_SUB_TILE = 256
_SUBS_PER_STEP = 2
_ROW_TILE = _SUB_TILE * _SUBS_PER_STEP


def _sub_tile(x_t, seg_c, seg_r, k, v, wq, bq, ws, bs, out_ref, *,
              num_segments, scale):
    q = (jnp.dot(x_t, wq, preferred_element_type=jnp.float32) + bq) * scale

    s = lax.dot_general(q.astype(jnp.bfloat16), k,
                        (((1,), (1,)), ((), ())),
                        preferred_element_type=jnp.float32)
    mask = seg_c == seg_r
    s = jnp.where(mask, s, -jnp.inf)
    m = jnp.max(s, axis=1, keepdims=True)
    p = jnp.exp(s - m)
    denom = jnp.sum(p, axis=1, keepdims=True)

    o = jnp.dot(p.astype(jnp.bfloat16), v,
                preferred_element_type=jnp.float32) * (1.0 / denom)
    o = o + jnp.dot(x_t, ws, preferred_element_type=jnp.float32) + bs

    first = jnp.min(seg_c)
    last = jnp.max(seg_c)
    for b in range(num_segments):
        @pl.when((b >= first) & (b <= last))
        def _pool():
            mb = seg_c == b
            pb = jnp.max(jnp.where(mb, o, -jnp.inf), axis=0,
                         keepdims=True)
            out_ref[b:b + 1, :] = jnp.maximum(out_ref[b:b + 1, :], pb)


def _atten_pool_kernel(x_full_ref, x_tile_ref, segc_ref, segr_ref,
                       wq_ref, bq_ref, wk_ref, bk_ref, wv_ref, bv_ref,
                       ws_ref, bs_ref,
                       out_ref, k_ref, v_ref, *, num_segments, scale):
    i = pl.program_id(0)

    @pl.when(i == 0)
    def _init():
        x_full = x_full_ref[:]
        k = jnp.dot(x_full, wk_ref[:],
                    preferred_element_type=jnp.float32) + bk_ref[:]
        v = jnp.dot(x_full, wv_ref[:],
                    preferred_element_type=jnp.float32) + bv_ref[:]
        k_ref[:] = k.astype(jnp.bfloat16)
        v_ref[:] = v.astype(jnp.bfloat16)
        out_ref[:] = jnp.full_like(out_ref, -jnp.inf)

    seg_r = segr_ref[:]
    k = k_ref[:]
    v = v_ref[:]
    for u in range(_SUBS_PER_STEP):
        lo = u * _SUB_TILE
        _sub_tile(x_tile_ref[lo:lo + _SUB_TILE, :],
                  segc_ref[0, lo:lo + _SUB_TILE, :], seg_r, k, v,
                  wq_ref[:], bq_ref[:], ws_ref[:], bs_ref[:], out_ref,
                  num_segments=num_segments, scale=scale)


def kernel(x, subgbatch, Wq, bq, Wk, bk, Wv, bv, Wskip, bskip):
    n, d = x.shape
    c = Wq.shape[1]
    num_segments = 16
    t = _ROW_TILE
    num_tiles = n // t
    seg = subgbatch.astype(jnp.int32)
    segc = seg.reshape(num_tiles, t, 1)
    segr = seg.reshape(1, n)

    fn = pl.pallas_call(
        functools.partial(_atten_pool_kernel, num_segments=num_segments,
                          scale=1.0 / math.sqrt(c)),
        grid=(num_tiles,),
        in_specs=[
            pl.BlockSpec((n, d), lambda i: (0, 0)),
            pl.BlockSpec((t, d), lambda i: (i, 0)),
            pl.BlockSpec((1, t, 1), lambda i: (i, 0, 0)),
            pl.BlockSpec((1, n), lambda i: (0, 0)),
            pl.BlockSpec((d, c), lambda i: (0, 0)),
            pl.BlockSpec((1, c), lambda i: (0, 0)),
            pl.BlockSpec((d, c), lambda i: (0, 0)),
            pl.BlockSpec((1, c), lambda i: (0, 0)),
            pl.BlockSpec((d, c), lambda i: (0, 0)),
            pl.BlockSpec((1, c), lambda i: (0, 0)),
            pl.BlockSpec((d, c), lambda i: (0, 0)),
            pl.BlockSpec((1, c), lambda i: (0, 0)),
        ],
        out_specs=pl.BlockSpec((num_segments, c), lambda i: (0, 0)),
        scratch_shapes=[
            pltpu.VMEM((n, c), jnp.bfloat16),
            pltpu.VMEM((n, c), jnp.bfloat16),
        ],
        out_shape=jax.ShapeDtypeStruct((num_segments, c), jnp.float32),
    )
    return fn(x, x, segc, segr,
              Wq, bq.reshape(1, c), Wk, bk.reshape(1, c),
              Wv, bv.reshape(1, c), Wskip, bskip.reshape(1, c))
